# Initial kernel scaffold; baseline (speedup 1.0000x reference)
import jax
import jax.numpy as jnp
from jax.experimental import pallas as pl


def kernel(hembs_0, hembs_1, hadjs_0, hadjs_1, w_g0_l0, asrc_g0_l0, atrg_g0_l0, w_g0_l1, asrc_g0_l1, atrg_g0_l1, w_g1_l0, asrc_g1_l0, atrg_g1_l0, w_g1_l1, asrc_g1_l1, atrg_g1_l1, fc_w, fc_b):
    raise NotImplementedError("write your pallas kernel here")


# speedup vs baseline: 48.5476x; 48.5476x over previous
import functools
import jax
import jax.numpy as jnp
from jax import lax
from jax.experimental import pallas as pl
from jax.experimental.pallas import tpu as pltpu
from jax.experimental.pallas import tpu_sc as plsc

NN = 10000
EE = 320000
ROW = 144
TROW = 16
BLK = 1000
GRID = NN // BLK
NSC = 2
NTILE = 16
NW = NSC * NTILE
EPT = EE // NW
CHUNK = 80
CPT = EPT // CHUNK
NPAD = 10240
RPT = NPAD // NTILE

_GDN = lax.GatherDimensionNumbers(
    offset_dims=(), collapsed_slice_dims=(0,), start_index_map=(0,))


def _take16(v, j):
  idx = jnp.full((16, 1), j, dtype=jnp.int32)
  return lax.gather(v, idx, _GDN, (1,),
                    mode=lax.GatherScatterMode.PROMISE_IN_BOUNDS)


def _leaky(x):
  return jnp.where(x >= 0, x, 0.2 * x)


def _proj_block(h, wcat_ref, asrc_ref, atrg_ref):
  hp = jnp.dot(h, wcat_ref[...], preferred_element_type=jnp.float32)
  a_s = asrc_ref[...]
  a_t = atrg_ref[...]
  s0 = jnp.dot(hp[:, :64], a_s[0][:, None], preferred_element_type=jnp.float32)
  s1 = jnp.dot(hp[:, 64:], a_s[1][:, None], preferred_element_type=jnp.float32)
  t0 = jnp.dot(hp[:, :64], a_t[0][:, None], preferred_element_type=jnp.float32)
  t1 = jnp.dot(hp[:, 64:], a_t[1][:, None], preferred_element_type=jnp.float32)
  zpad = jnp.zeros((h.shape[0], ROW - 130), jnp.float32)
  hpa = jnp.concatenate([hp, s0, s1, zpad], axis=1)
  tpad = jnp.zeros((h.shape[0], TROW - 2), jnp.float32)
  atrgp = jnp.concatenate([t0, t1, tpad], axis=1)
  return hpa, atrgp, jnp.max(jnp.maximum(s0, s1)), jnp.max(jnp.maximum(t0, t1))


def _update_max(mx_ref, ms, mt):
  i = pl.program_id(0)

  @pl.when(i == 0)
  def _():
    mx_ref[...] = jnp.full((1, 16), -jnp.inf, jnp.float32)

  lanes = lax.broadcasted_iota(jnp.int32, (1, 16), 1)
  vec = jnp.where(lanes == 0, ms, jnp.where(lanes == 1, mt, -jnp.inf))
  mx_ref[...] = jnp.maximum(mx_ref[...], vec)


def _dense_body(h_ref, wcat_ref, asrc_ref, atrg_ref,
                hpa_ref, atrgp_ref, mx_ref):
  hpa, atrgp, ms, mt = _proj_block(h_ref[...], wcat_ref, asrc_ref, atrg_ref)
  hpa_ref[...] = hpa
  atrgp_ref[...] = atrgp
  _update_max(mx_ref, ms, mt)


def _dense(h, wcat, asrc, atrg):
  return pl.pallas_call(
      _dense_body,
      grid=(GRID,),
      in_specs=[
          pl.BlockSpec((BLK, 128), lambda i: (i, 0)),
          pl.BlockSpec((128, 128), lambda i: (0, 0)),
          pl.BlockSpec((2, 64), lambda i: (0, 0)),
          pl.BlockSpec((2, 64), lambda i: (0, 0)),
      ],
      out_specs=[
          pl.BlockSpec((BLK, ROW), lambda i: (i, 0)),
          pl.BlockSpec((BLK, TROW), lambda i: (i, 0)),
          pl.BlockSpec((1, 16), lambda i: (0, 0)),
      ],
      out_shape=[
          jax.ShapeDtypeStruct((NN, ROW), jnp.float32),
          jax.ShapeDtypeStruct((NN, TROW), jnp.float32),
          jax.ShapeDtypeStruct((1, 16), jnp.float32),
      ],
  )(h, wcat, asrc, atrg)


def _normalize(acc_ref):
  a = acc_ref[0] + acc_ref[1]
  d0 = a[:, 128:129] + 1e-16
  d1 = a[:, 129:130] + 1e-16
  return jnp.concatenate([a[:, :64] / d0, a[:, 64:128] / d1], axis=1)


def _combine_body(acc_ref, wcat_ref, asrc_ref, atrg_ref,
                  hpa_ref, atrgp_ref, mx_ref):
  x = _normalize(acc_ref)
  emb = jnp.where(x > 0, x, jnp.exp(x) - 1.0)
  hpa, atrgp, ms, mt = _proj_block(emb, wcat_ref, asrc_ref, atrg_ref)
  hpa_ref[...] = hpa
  atrgp_ref[...] = atrgp
  _update_max(mx_ref, ms, mt)


def _combine(acc, wcat, asrc, atrg):
  return pl.pallas_call(
      _combine_body,
      grid=(GRID,),
      in_specs=[
          pl.BlockSpec((2, BLK, ROW), lambda i: (0, i, 0)),
          pl.BlockSpec((128, 128), lambda i: (0, 0)),
          pl.BlockSpec((2, 64), lambda i: (0, 0)),
          pl.BlockSpec((2, 64), lambda i: (0, 0)),
      ],
      out_specs=[
          pl.BlockSpec((BLK, ROW), lambda i: (i, 0)),
          pl.BlockSpec((BLK, TROW), lambda i: (i, 0)),
          pl.BlockSpec((1, 16), lambda i: (0, 0)),
      ],
      out_shape=[
          jax.ShapeDtypeStruct((NN, ROW), jnp.float32),
          jax.ShapeDtypeStruct((NN, TROW), jnp.float32),
          jax.ShapeDtypeStruct((1, 16), jnp.float32),
      ],
  )(acc, wcat, asrc, atrg)


def _final_body(acc0_ref, acc1_ref, fcw_ref, fcb_ref, out_ref):
  x0 = _normalize(acc0_ref)
  x1 = _normalize(acc1_ref)
  e0 = (x0[:, :64] + x0[:, 64:]) * 0.5
  e1 = (x1[:, :64] + x1[:, 64:]) * 0.5
  w = fcw_ref[...]
  logits = (jnp.dot(e0, w[:64], preferred_element_type=jnp.float32)
            + jnp.dot(e1, w[64:], preferred_element_type=jnp.float32)
            + fcb_ref[...])
  m = jnp.max(logits, axis=-1, keepdims=True)
  z = logits - m
  lse = jnp.log(jnp.sum(jnp.exp(z), axis=-1, keepdims=True))
  out_ref[...] = z - lse


def _final(acc0, acc1, fc_w, fc_b):
  return pl.pallas_call(
      _final_body,
      grid=(GRID,),
      in_specs=[
          pl.BlockSpec((2, BLK, ROW), lambda i: (0, i, 0)),
          pl.BlockSpec((2, BLK, ROW), lambda i: (0, i, 0)),
          pl.BlockSpec((128, 4), lambda i: (0, 0)),
          pl.BlockSpec((1, 4), lambda i: (0, 0)),
      ],
      out_specs=pl.BlockSpec((BLK, 4), lambda i: (i, 0)),
      out_shape=jax.ShapeDtypeStruct((NN, 4), jnp.float32),
  )(acc0, acc1, fc_w, fc_b)


def _edge_body(hpa_hbm, atrgp_hbm, mx_hbm, src_hbm, trg_hbm, zero_hbm, out_hbm,
               si, ti, rows, trows, orows, mxv, acc):
  c = lax.axis_index("c")
  s = lax.axis_index("s")
  wid = s * NSC + c

  pltpu.sync_copy(zero_hbm, acc.at[pl.ds(s * RPT, RPT)])
  plsc.subcore_barrier()

  pltpu.sync_copy(mx_hbm.at[0], mxv)
  mvec = mxv[...]
  msum = _take16(mvec, 0) + _take16(mvec, 1)
  mshift = _leaky(msum)

  base0 = wid * EPT

  def chunk_body(k, carry):
    base = base0 + k * CHUNK
    pltpu.sync_copy(src_hbm.at[pl.ds(base, CHUNK)], si)
    pltpu.sync_copy(trg_hbm.at[pl.ds(base, CHUNK)], ti)
    pltpu.sync_copy(hpa_hbm.at[si], rows)
    pltpu.sync_copy(atrgp_hbm.at[ti], trows)

    for g in range(CHUNK // 16):
      ridx = jnp.full((16,), g * 16, jnp.int32) + lax.iota(jnp.int32, 16)
      c128 = jnp.full((16,), 128, jnp.int32)
      c129 = jnp.full((16,), 129, jnp.int32)
      s0 = plsc.load_gather(rows, [ridx, c128])
      s1 = plsc.load_gather(rows, [ridx, c129])
      t0 = plsc.load_gather(trows, [ridx, jnp.zeros((16,), jnp.int32)])
      t1 = plsc.load_gather(trows, [ridx, jnp.full((16,), 1, jnp.int32)])
      c0 = jnp.exp(_leaky(s0 + t0) - mshift)
      c1 = jnp.exp(_leaky(s1 + t1) - mshift)
      plsc.store_scatter(orows, [ridx, c128], c0)
      plsc.store_scatter(orows, [ridx, c129], c1)
      for j in range(16):
        e = g * 16 + j
        b0 = _take16(c0, j)
        b1 = _take16(c1, j)
        for q in range(4):
          orows[e, pl.ds(q * 16, 16)] = rows[e, pl.ds(q * 16, 16)] * b0
        for q in range(4):
          orows[e, pl.ds(64 + q * 16, 16)] = rows[e, pl.ds(64 + q * 16, 16)] * b1

    pltpu.sync_copy(orows, acc.at[ti], add=True)
    return carry

  lax.fori_loop(0, CPT, chunk_body, 0)

  plsc.subcore_barrier()
  pltpu.sync_copy(acc.at[pl.ds(s * RPT, RPT)],
                  out_hbm.at[c, pl.ds(s * RPT, RPT)])


_edge = pl.kernel(
    _edge_body,
    out_type=jax.ShapeDtypeStruct((NSC, NPAD, ROW), jnp.float32),
    mesh=plsc.VectorSubcoreMesh(core_axis_name="c", subcore_axis_name="s"),
    compiler_params=pltpu.CompilerParams(use_tc_tiling_on_sc=False, needs_layout_passes=False),
    scratch_types=[
        pltpu.VMEM((CHUNK,), jnp.int32),
        pltpu.VMEM((CHUNK,), jnp.int32),
        pltpu.VMEM((CHUNK, ROW), jnp.float32),
        pltpu.VMEM((CHUNK, TROW), jnp.float32),
        pltpu.VMEM((CHUNK, ROW), jnp.float32),
        pltpu.VMEM((16,), jnp.float32),
        pltpu.VMEM_SHARED((NPAD, ROW), jnp.float32),
    ],
)


def _prep(w, a_s, a_t):
  wcat = jnp.transpose(w, (1, 0, 2)).reshape(128, 128)
  return wcat, a_s[:, :, 0], a_t[:, :, 0]


def _stack(h, adj, zeros_hbm, w0, s0, t0, w1, s1, t1):
  wcat0, av0, bv0 = _prep(w0, s0, t0)
  wcat1, av1, bv1 = _prep(w1, s1, t1)
  hpa, atrgp, mx = _dense(h, wcat0, av0, bv0)
  acc = _edge(hpa, atrgp, mx, adj[0], adj[1], zeros_hbm)
  hpa1, atrgp1, mx1 = _combine(acc, wcat1, av1, bv1)
  acc1 = _edge(hpa1, atrgp1, mx1, adj[0], adj[1], zeros_hbm)
  return acc1


def kernel(hembs_0, hembs_1, hadjs_0, hadjs_1, w_g0_l0, asrc_g0_l0, atrg_g0_l0,
           w_g0_l1, asrc_g0_l1, atrg_g0_l1, w_g1_l0, asrc_g1_l0, atrg_g1_l0,
           w_g1_l1, asrc_g1_l1, atrg_g1_l1, fc_w, fc_b):
  zeros_hbm = jnp.zeros((RPT, ROW), jnp.float32)
  acc_g0 = _stack(hembs_0, hadjs_0, zeros_hbm,
                  w_g0_l0, asrc_g0_l0, atrg_g0_l0,
                  w_g0_l1, asrc_g0_l1, atrg_g0_l1)
  acc_g1 = _stack(hembs_1, hadjs_1, zeros_hbm,
                  w_g1_l0, asrc_g1_l0, atrg_g1_l0,
                  w_g1_l1, asrc_g1_l1, atrg_g1_l1)
  return _final(acc_g0, acc_g1, fc_w, fc_b.reshape(1, 4))


# speedup vs baseline: 65.7879x; 1.3551x over previous
import functools
import jax
import jax.numpy as jnp
from jax import lax
from jax.experimental import pallas as pl
from jax.experimental.pallas import tpu as pltpu
from jax.experimental.pallas import tpu_sc as plsc

NN = 10000
EE = 320000
ROW = 144
TROW = 16
BLK = 1000
GRID = NN // BLK
NSC = 2
NTILE = 16
NW = NSC * NTILE
EPT = EE // NW
CHUNK = 80
CPT = EPT // CHUNK
NPAD = 10240
RPT = NPAD // NTILE

_GDN = lax.GatherDimensionNumbers(
    offset_dims=(), collapsed_slice_dims=(0,), start_index_map=(0,))


def _take16(v, j):
  idx = jnp.full((16, 1), j, dtype=jnp.int32)
  return lax.gather(v, idx, _GDN, (1,),
                    mode=lax.GatherScatterMode.PROMISE_IN_BOUNDS)


def _leaky(x):
  return jnp.where(x >= 0, x, 0.2 * x)


def _proj_block(h, wcat_ref, asrc_ref, atrg_ref):
  hp = jnp.dot(h, wcat_ref[...], preferred_element_type=jnp.float32)
  a_s = asrc_ref[...]
  a_t = atrg_ref[...]
  s0 = jnp.dot(hp[:, :64], a_s[0][:, None], preferred_element_type=jnp.float32)
  s1 = jnp.dot(hp[:, 64:], a_s[1][:, None], preferred_element_type=jnp.float32)
  t0 = jnp.dot(hp[:, :64], a_t[0][:, None], preferred_element_type=jnp.float32)
  t1 = jnp.dot(hp[:, 64:], a_t[1][:, None], preferred_element_type=jnp.float32)
  zpad = jnp.zeros((h.shape[0], ROW - 130), jnp.float32)
  hpa = jnp.concatenate([hp, s0, s1, zpad], axis=1)
  tpad = jnp.zeros((h.shape[0], TROW - 2), jnp.float32)
  atrgp = jnp.concatenate([t0, t1, tpad], axis=1)
  return hpa, atrgp, jnp.max(jnp.maximum(s0, s1)), jnp.max(jnp.maximum(t0, t1))


def _update_max(mx_ref, ms, mt):
  i = pl.program_id(0)

  @pl.when(i == 0)
  def _():
    mx_ref[...] = jnp.full((1, 16), -jnp.inf, jnp.float32)

  lanes = lax.broadcasted_iota(jnp.int32, (1, 16), 1)
  vec = jnp.where(lanes == 0, ms, jnp.where(lanes == 1, mt, -jnp.inf))
  mx_ref[...] = jnp.maximum(mx_ref[...], vec)


def _dense_body(h_ref, wcat_ref, asrc_ref, atrg_ref,
                hpa_ref, atrgp_ref, mx_ref):
  hpa, atrgp, ms, mt = _proj_block(h_ref[...], wcat_ref, asrc_ref, atrg_ref)
  hpa_ref[...] = hpa
  atrgp_ref[...] = atrgp
  _update_max(mx_ref, ms, mt)


def _dense(h, wcat, asrc, atrg):
  return pl.pallas_call(
      _dense_body,
      grid=(GRID,),
      in_specs=[
          pl.BlockSpec((BLK, 128), lambda i: (i, 0)),
          pl.BlockSpec((128, 128), lambda i: (0, 0)),
          pl.BlockSpec((2, 64), lambda i: (0, 0)),
          pl.BlockSpec((2, 64), lambda i: (0, 0)),
      ],
      out_specs=[
          pl.BlockSpec((BLK, ROW), lambda i: (i, 0)),
          pl.BlockSpec((BLK, TROW), lambda i: (i, 0)),
          pl.BlockSpec((1, 16), lambda i: (0, 0)),
      ],
      out_shape=[
          jax.ShapeDtypeStruct((NN, ROW), jnp.float32),
          jax.ShapeDtypeStruct((NN, TROW), jnp.float32),
          jax.ShapeDtypeStruct((1, 16), jnp.float32),
      ],
  )(h, wcat, asrc, atrg)


def _normalize(acc_ref):
  a = acc_ref[0] + acc_ref[1]
  d0 = a[:, 128:129] + 1e-16
  d1 = a[:, 129:130] + 1e-16
  return jnp.concatenate([a[:, :64] / d0, a[:, 64:128] / d1], axis=1)


def _combine_body(acc_ref, wcat_ref, asrc_ref, atrg_ref,
                  hpa_ref, atrgp_ref, mx_ref):
  x = _normalize(acc_ref)
  emb = jnp.where(x > 0, x, jnp.exp(x) - 1.0)
  hpa, atrgp, ms, mt = _proj_block(emb, wcat_ref, asrc_ref, atrg_ref)
  hpa_ref[...] = hpa
  atrgp_ref[...] = atrgp
  _update_max(mx_ref, ms, mt)


def _combine(acc, wcat, asrc, atrg):
  return pl.pallas_call(
      _combine_body,
      grid=(GRID,),
      in_specs=[
          pl.BlockSpec((2, BLK, ROW), lambda i: (0, i, 0)),
          pl.BlockSpec((128, 128), lambda i: (0, 0)),
          pl.BlockSpec((2, 64), lambda i: (0, 0)),
          pl.BlockSpec((2, 64), lambda i: (0, 0)),
      ],
      out_specs=[
          pl.BlockSpec((BLK, ROW), lambda i: (i, 0)),
          pl.BlockSpec((BLK, TROW), lambda i: (i, 0)),
          pl.BlockSpec((1, 16), lambda i: (0, 0)),
      ],
      out_shape=[
          jax.ShapeDtypeStruct((NN, ROW), jnp.float32),
          jax.ShapeDtypeStruct((NN, TROW), jnp.float32),
          jax.ShapeDtypeStruct((1, 16), jnp.float32),
      ],
  )(acc, wcat, asrc, atrg)


def _final_body(acc0_ref, acc1_ref, fcw_ref, fcb_ref, out_ref):
  x0 = _normalize(acc0_ref)
  x1 = _normalize(acc1_ref)
  e0 = (x0[:, :64] + x0[:, 64:]) * 0.5
  e1 = (x1[:, :64] + x1[:, 64:]) * 0.5
  w = fcw_ref[...]
  logits = (jnp.dot(e0, w[:64], preferred_element_type=jnp.float32)
            + jnp.dot(e1, w[64:], preferred_element_type=jnp.float32)
            + fcb_ref[...])
  m = jnp.max(logits, axis=-1, keepdims=True)
  z = logits - m
  lse = jnp.log(jnp.sum(jnp.exp(z), axis=-1, keepdims=True))
  out_ref[...] = z - lse


def _final(acc0, acc1, fc_w, fc_b):
  return pl.pallas_call(
      _final_body,
      grid=(GRID,),
      in_specs=[
          pl.BlockSpec((2, BLK, ROW), lambda i: (0, i, 0)),
          pl.BlockSpec((2, BLK, ROW), lambda i: (0, i, 0)),
          pl.BlockSpec((128, 4), lambda i: (0, 0)),
          pl.BlockSpec((1, 4), lambda i: (0, 0)),
      ],
      out_specs=pl.BlockSpec((BLK, 4), lambda i: (i, 0)),
      out_shape=jax.ShapeDtypeStruct((NN, 4), jnp.float32),
  )(acc0, acc1, fc_w, fc_b)


def _edge_body(hpa_hbm, atrgp_hbm, mx_hbm, src_hbm, trg_hbm, zero_hbm, out_hbm,
               si, ti, rows, trows, orows, mxv, sem1, sem2, acc):
  c = lax.axis_index("c")
  s = lax.axis_index("s")
  wid = s * NSC + c

  pltpu.sync_copy(zero_hbm, acc.at[pl.ds(s * RPT, RPT)])
  plsc.subcore_barrier()

  pltpu.sync_copy(mx_hbm.at[0], mxv)
  mvec = mxv[...]
  msum = _take16(mvec, 0) + _take16(mvec, 1)
  mshift = _leaky(msum)

  base0 = wid * EPT

  def chunk_body(k, carry):
    base = base0 + k * CHUNK
    c1 = pltpu.async_copy(src_hbm.at[pl.ds(base, CHUNK)], si, sem1)
    c2 = pltpu.async_copy(trg_hbm.at[pl.ds(base, CHUNK)], ti, sem2)
    c1.wait()
    c2.wait()
    c1 = pltpu.async_copy(hpa_hbm.at[si], rows, sem1)
    c2 = pltpu.async_copy(atrgp_hbm.at[ti], trows, sem2)
    c1.wait()
    c2.wait()

    for g in range(CHUNK // 16):
      ridx = jnp.full((16,), g * 16, jnp.int32) + lax.iota(jnp.int32, 16)
      c128 = jnp.full((16,), 128, jnp.int32)
      c129 = jnp.full((16,), 129, jnp.int32)
      s0 = plsc.load_gather(rows, [ridx, c128])
      s1 = plsc.load_gather(rows, [ridx, c129])
      t0 = plsc.load_gather(trows, [ridx, jnp.zeros((16,), jnp.int32)])
      t1 = plsc.load_gather(trows, [ridx, jnp.full((16,), 1, jnp.int32)])
      c0 = jnp.exp(_leaky(s0 + t0) - mshift)
      c1 = jnp.exp(_leaky(s1 + t1) - mshift)
      plsc.store_scatter(orows, [ridx, c128], c0)
      plsc.store_scatter(orows, [ridx, c129], c1)
      for j in range(16):
        e = g * 16 + j
        b0 = _take16(c0, j)
        b1 = _take16(c1, j)
        for q in range(4):
          orows[e, pl.ds(q * 16, 16)] = rows[e, pl.ds(q * 16, 16)] * b0
        for q in range(4):
          orows[e, pl.ds(64 + q * 16, 16)] = rows[e, pl.ds(64 + q * 16, 16)] * b1

    pltpu.sync_copy(orows, acc.at[ti], add=True)
    return carry

  lax.fori_loop(0, CPT, chunk_body, 0)

  plsc.subcore_barrier()
  pltpu.sync_copy(acc.at[pl.ds(s * RPT, RPT)],
                  out_hbm.at[c, pl.ds(s * RPT, RPT)])


_edge = pl.kernel(
    _edge_body,
    out_type=jax.ShapeDtypeStruct((NSC, NPAD, ROW), jnp.float32),
    mesh=plsc.VectorSubcoreMesh(core_axis_name="c", subcore_axis_name="s"),
    compiler_params=pltpu.CompilerParams(use_tc_tiling_on_sc=False, needs_layout_passes=False),
    scratch_types=[
        pltpu.VMEM((CHUNK,), jnp.int32),
        pltpu.VMEM((CHUNK,), jnp.int32),
        pltpu.VMEM((CHUNK, ROW), jnp.float32),
        pltpu.VMEM((CHUNK, TROW), jnp.float32),
        pltpu.VMEM((CHUNK, ROW), jnp.float32),
        pltpu.VMEM((16,), jnp.float32),
        pltpu.SemaphoreType.DMA,
        pltpu.SemaphoreType.DMA,
        pltpu.VMEM_SHARED((NPAD, ROW), jnp.float32),
    ],
)


def _prep(w, a_s, a_t):
  wcat = jnp.transpose(w, (1, 0, 2)).reshape(128, 128)
  return wcat, a_s[:, :, 0], a_t[:, :, 0]


def _stack(h, adj, zeros_hbm, w0, s0, t0, w1, s1, t1):
  wcat0, av0, bv0 = _prep(w0, s0, t0)
  wcat1, av1, bv1 = _prep(w1, s1, t1)
  hpa, atrgp, mx = _dense(h, wcat0, av0, bv0)
  acc = _edge(hpa, atrgp, mx, adj[0], adj[1], zeros_hbm)
  hpa1, atrgp1, mx1 = _combine(acc, wcat1, av1, bv1)
  acc1 = _edge(hpa1, atrgp1, mx1, adj[0], adj[1], zeros_hbm)
  return acc1


def kernel(hembs_0, hembs_1, hadjs_0, hadjs_1, w_g0_l0, asrc_g0_l0, atrg_g0_l0,
           w_g0_l1, asrc_g0_l1, atrg_g0_l1, w_g1_l0, asrc_g1_l0, atrg_g1_l0,
           w_g1_l1, asrc_g1_l1, atrg_g1_l1, fc_w, fc_b):
  zeros_hbm = jnp.zeros((RPT, ROW), jnp.float32)
  acc_g0 = _stack(hembs_0, hadjs_0, zeros_hbm,
                  w_g0_l0, asrc_g0_l0, atrg_g0_l0,
                  w_g0_l1, asrc_g0_l1, atrg_g0_l1)
  acc_g1 = _stack(hembs_1, hadjs_1, zeros_hbm,
                  w_g1_l0, asrc_g1_l0, atrg_g1_l0,
                  w_g1_l1, asrc_g1_l1, atrg_g1_l1)
  return _final(acc_g0, acc_g1, fc_w, fc_b.reshape(1, 4))


# speedup vs baseline: 65.9691x; 1.0028x over previous
import jax
import jax.numpy as jnp
from jax import lax
from jax.experimental import pallas as pl
from jax.experimental.pallas import tpu as pltpu
from jax.experimental.pallas import tpu_sc as plsc

NN = 10000
EE = 320000
ROW = 144
TROW = 16
BLK = 1000
GRID = NN // BLK
NSC = 2
NTILE = 16
NW = NSC * NTILE
EPT = EE // NW
CHUNK = 80
CPT = EPT // CHUNK
NPAD = 10240
RPT = NPAD // NTILE

_GDN = lax.GatherDimensionNumbers(
    offset_dims=(), collapsed_slice_dims=(0,), start_index_map=(0,))


def _take16(v, j):
  idx = jnp.full((16, 1), j, dtype=jnp.int32)
  return lax.gather(v, idx, _GDN, (1,),
                    mode=lax.GatherScatterMode.PROMISE_IN_BOUNDS)


def _leaky(x):
  return jnp.where(x >= 0, x, 0.2 * x)


def _proj_block(h, wcat_ref, asrc_ref, atrg_ref):
  hp = jnp.dot(h, wcat_ref[...], preferred_element_type=jnp.float32)
  a_s = asrc_ref[...]
  a_t = atrg_ref[...]
  s0 = jnp.dot(hp[:, :64], a_s[0][:, None], preferred_element_type=jnp.float32)
  s1 = jnp.dot(hp[:, 64:], a_s[1][:, None], preferred_element_type=jnp.float32)
  t0 = jnp.dot(hp[:, :64], a_t[0][:, None], preferred_element_type=jnp.float32)
  t1 = jnp.dot(hp[:, 64:], a_t[1][:, None], preferred_element_type=jnp.float32)
  zpad = jnp.zeros((h.shape[0], ROW - 130), jnp.float32)
  hpa = jnp.concatenate([hp, s0, s1, zpad], axis=1)
  tpad = jnp.zeros((h.shape[0], TROW - 2), jnp.float32)
  atrgp = jnp.concatenate([t0, t1, tpad], axis=1)
  return hpa, atrgp, jnp.max(jnp.maximum(s0, s1)), jnp.max(jnp.maximum(t0, t1))


def _update_max(mx_ref, ms, mt):
  i = pl.program_id(0)

  @pl.when(i == 0)
  def _():
    mx_ref[...] = jnp.full((1, 16), -jnp.inf, jnp.float32)

  lanes = lax.broadcasted_iota(jnp.int32, (1, 16), 1)
  vec = jnp.where(lanes == 0, ms, jnp.where(lanes == 1, mt, -jnp.inf))
  mx_ref[...] = jnp.maximum(mx_ref[...], vec)


def _dense_body(h_ref, wcat_ref, asrc_ref, atrg_ref,
                hpa_ref, atrgp_ref, mx_ref):
  hpa, atrgp, ms, mt = _proj_block(h_ref[...], wcat_ref, asrc_ref, atrg_ref)
  hpa_ref[...] = hpa
  atrgp_ref[...] = atrgp
  _update_max(mx_ref, ms, mt)


def _dense(h, wcat, asrc, atrg):
  return pl.pallas_call(
      _dense_body,
      grid=(GRID,),
      in_specs=[
          pl.BlockSpec((BLK, 128), lambda i: (i, 0)),
          pl.BlockSpec((128, 128), lambda i: (0, 0)),
          pl.BlockSpec((2, 64), lambda i: (0, 0)),
          pl.BlockSpec((2, 64), lambda i: (0, 0)),
      ],
      out_specs=[
          pl.BlockSpec((BLK, ROW), lambda i: (i, 0)),
          pl.BlockSpec((BLK, TROW), lambda i: (i, 0)),
          pl.BlockSpec((1, 16), lambda i: (0, 0)),
      ],
      out_shape=[
          jax.ShapeDtypeStruct((NN, ROW), jnp.float32),
          jax.ShapeDtypeStruct((NN, TROW), jnp.float32),
          jax.ShapeDtypeStruct((1, 16), jnp.float32),
      ],
  )(h, wcat, asrc, atrg)


def _normalize(acc_ref):
  a = acc_ref[0] + acc_ref[1]
  d0 = a[:, 128:129] + 1e-16
  d1 = a[:, 129:130] + 1e-16
  return jnp.concatenate([a[:, :64] / d0, a[:, 64:128] / d1], axis=1)


def _combine_body(acc_ref, wcat_ref, asrc_ref, atrg_ref,
                  hpa_ref, atrgp_ref, mx_ref):
  x = _normalize(acc_ref)
  emb = jnp.where(x > 0, x, jnp.exp(x) - 1.0)
  hpa, atrgp, ms, mt = _proj_block(emb, wcat_ref, asrc_ref, atrg_ref)
  hpa_ref[...] = hpa
  atrgp_ref[...] = atrgp
  _update_max(mx_ref, ms, mt)


def _combine(acc, wcat, asrc, atrg):
  return pl.pallas_call(
      _combine_body,
      grid=(GRID,),
      in_specs=[
          pl.BlockSpec((2, BLK, ROW), lambda i: (0, i, 0)),
          pl.BlockSpec((128, 128), lambda i: (0, 0)),
          pl.BlockSpec((2, 64), lambda i: (0, 0)),
          pl.BlockSpec((2, 64), lambda i: (0, 0)),
      ],
      out_specs=[
          pl.BlockSpec((BLK, ROW), lambda i: (i, 0)),
          pl.BlockSpec((BLK, TROW), lambda i: (i, 0)),
          pl.BlockSpec((1, 16), lambda i: (0, 0)),
      ],
      out_shape=[
          jax.ShapeDtypeStruct((NN, ROW), jnp.float32),
          jax.ShapeDtypeStruct((NN, TROW), jnp.float32),
          jax.ShapeDtypeStruct((1, 16), jnp.float32),
      ],
  )(acc, wcat, asrc, atrg)


def _final_body(acc0_ref, acc1_ref, fcw_ref, fcb_ref, out_ref):
  x0 = _normalize(acc0_ref)
  x1 = _normalize(acc1_ref)
  e0 = (x0[:, :64] + x0[:, 64:]) * 0.5
  e1 = (x1[:, :64] + x1[:, 64:]) * 0.5
  w = fcw_ref[...]
  logits = (jnp.dot(e0, w[:64], preferred_element_type=jnp.float32)
            + jnp.dot(e1, w[64:], preferred_element_type=jnp.float32)
            + fcb_ref[...])
  m = jnp.max(logits, axis=-1, keepdims=True)
  z = logits - m
  lse = jnp.log(jnp.sum(jnp.exp(z), axis=-1, keepdims=True))
  out_ref[...] = z - lse


def _final(acc0, acc1, fc_w, fc_b):
  return pl.pallas_call(
      _final_body,
      grid=(GRID,),
      in_specs=[
          pl.BlockSpec((2, BLK, ROW), lambda i: (0, i, 0)),
          pl.BlockSpec((2, BLK, ROW), lambda i: (0, i, 0)),
          pl.BlockSpec((128, 4), lambda i: (0, 0)),
          pl.BlockSpec((1, 4), lambda i: (0, 0)),
      ],
      out_specs=pl.BlockSpec((BLK, 4), lambda i: (i, 0)),
      out_shape=jax.ShapeDtypeStruct((NN, 4), jnp.float32),
  )(acc0, acc1, fc_w, fc_b)


def _edge_body(hpa_hbm, atrgp_hbm, mx_hbm, src_hbm, trg_hbm, zero_hbm, out_hbm,
               si, ti, rows, trows, orows, mxv, sem1, sem2, acc):
  c = lax.axis_index("c")
  s = lax.axis_index("s")
  wid = s * NSC + c

  pltpu.sync_copy(zero_hbm, acc.at[pl.ds(s * RPT, RPT)])
  plsc.subcore_barrier()

  pltpu.sync_copy(mx_hbm.at[0], mxv)
  mvec = mxv[...]
  msum = _take16(mvec, 0) + _take16(mvec, 1)
  mshift = _leaky(msum)

  base0 = wid * EPT

  def chunk_body(k, carry):
    base = base0 + k * CHUNK
    c1 = pltpu.async_copy(src_hbm.at[pl.ds(base, CHUNK)], si, sem1)
    c2 = pltpu.async_copy(trg_hbm.at[pl.ds(base, CHUNK)], ti, sem2)
    c1.wait()
    c2.wait()
    c1 = pltpu.async_copy(hpa_hbm.at[si], rows, sem1)
    c2 = pltpu.async_copy(atrgp_hbm.at[ti], trows, sem2)
    c1.wait()
    c2.wait()

    for g in range(CHUNK // 16):
      ridx = jnp.full((16,), g * 16, jnp.int32) + lax.iota(jnp.int32, 16)
      c128 = jnp.full((16,), 128, jnp.int32)
      c129 = jnp.full((16,), 129, jnp.int32)
      s0 = plsc.load_gather(rows, [ridx, c128])
      s1 = plsc.load_gather(rows, [ridx, c129])
      t0 = plsc.load_gather(trows, [ridx, jnp.zeros((16,), jnp.int32)])
      t1 = plsc.load_gather(trows, [ridx, jnp.full((16,), 1, jnp.int32)])
      c0 = jnp.exp(_leaky(s0 + t0) - mshift)
      c1 = jnp.exp(_leaky(s1 + t1) - mshift)
      plsc.store_scatter(orows, [ridx, c128], c0)
      plsc.store_scatter(orows, [ridx, c129], c1)
      for j in range(16):
        e = g * 16 + j
        b0 = _take16(c0, j)
        b1 = _take16(c1, j)
        for q in range(4):
          orows[e, pl.ds(q * 16, 16)] = rows[e, pl.ds(q * 16, 16)] * b0
        for q in range(4):
          orows[e, pl.ds(64 + q * 16, 16)] = rows[e, pl.ds(64 + q * 16, 16)] * b1

    pltpu.sync_copy(orows, acc.at[ti], add=True)
    return carry

  lax.fori_loop(0, CPT, chunk_body, 0)

  plsc.subcore_barrier()
  pltpu.sync_copy(acc.at[pl.ds(s * RPT, RPT)],
                  out_hbm.at[c, pl.ds(s * RPT, RPT)])


_edge = pl.kernel(
    _edge_body,
    out_type=jax.ShapeDtypeStruct((NSC, NPAD, ROW), jnp.float32),
    mesh=plsc.VectorSubcoreMesh(core_axis_name="c", subcore_axis_name="s"),
    compiler_params=pltpu.CompilerParams(use_tc_tiling_on_sc=False, needs_layout_passes=False),
    scratch_types=[
        pltpu.VMEM((CHUNK,), jnp.int32),
        pltpu.VMEM((CHUNK,), jnp.int32),
        pltpu.VMEM((CHUNK, ROW), jnp.float32),
        pltpu.VMEM((CHUNK, TROW), jnp.float32),
        pltpu.VMEM((CHUNK, ROW), jnp.float32),
        pltpu.VMEM((16,), jnp.float32),
        pltpu.SemaphoreType.DMA,
        pltpu.SemaphoreType.DMA,
        pltpu.VMEM_SHARED((NPAD, ROW), jnp.float32),
    ],
)


def _prep(w, a_s, a_t):
  wcat = jnp.transpose(w, (1, 0, 2)).reshape(128, 128)
  return wcat, a_s[:, :, 0], a_t[:, :, 0]


def _stack(h, adj, zeros_hbm, w0, s0, t0, w1, s1, t1):
  wcat0, av0, bv0 = _prep(w0, s0, t0)
  wcat1, av1, bv1 = _prep(w1, s1, t1)
  hpa, atrgp, mx = _dense(h, wcat0, av0, bv0)
  acc = _edge(hpa, atrgp, mx, adj[0], adj[1], zeros_hbm)
  hpa1, atrgp1, mx1 = _combine(acc, wcat1, av1, bv1)
  acc1 = _edge(hpa1, atrgp1, mx1, adj[0], adj[1], zeros_hbm)
  return acc1


def kernel(hembs_0, hembs_1, hadjs_0, hadjs_1, w_g0_l0, asrc_g0_l0, atrg_g0_l0,
           w_g0_l1, asrc_g0_l1, atrg_g0_l1, w_g1_l0, asrc_g1_l0, atrg_g1_l0,
           w_g1_l1, asrc_g1_l1, atrg_g1_l1, fc_w, fc_b):
  zeros_hbm = jnp.zeros((RPT, ROW), jnp.float32)
  acc_g0 = _stack(hembs_0, hadjs_0, zeros_hbm,
                  w_g0_l0, asrc_g0_l0, atrg_g0_l0,
                  w_g0_l1, asrc_g0_l1, atrg_g0_l1)
  acc_g1 = _stack(hembs_1, hadjs_1, zeros_hbm,
                  w_g1_l0, asrc_g1_l0, atrg_g1_l0,
                  w_g1_l1, asrc_g1_l1, atrg_g1_l1)
  return _final(acc_g0, acc_g1, fc_w, fc_b.reshape(1, 4))
